# baseline (speedup 1.0000x reference)
import functools

import jax
import jax.numpy as jnp
from jax import lax
from jax.experimental import pallas as pl
from jax.experimental.pallas import tpu as pltpu
from jax.experimental.pallas import tpu_sc as plsc

D = 1024
NC = 2
NS = 16
NW = NC * NS
K = 32
LANES = 16


@functools.lru_cache(maxsize=None)
def _emb_call(n_rows, seq_len, n_chunks):
    mesh = plsc.VectorSubcoreMesh(core_axis_name="c", subcore_axis_name="s")
    rows_per_w = n_rows // NW

    @functools.partial(
        pl.kernel,
        out_type=jax.ShapeDtypeStruct((n_rows, D), jnp.float32),
        mesh=mesh,
        scratch_types=[
            pltpu.VMEM((n_chunks, K), jnp.int32),
            pltpu.VMEM((K, D), jnp.float32),
            pltpu.VMEM((K, D), jnp.float32),
            pltpu.SemaphoreType.DMA,
        ],
    )
    def k(ids_hbm, table_hbm, pe_hbm, out_hbm, idx_v, rows_v, pe_v, sem):
        wid = lax.axis_index("s") * NC + lax.axis_index("c")
        base = wid * rows_per_w
        s_base = lax.rem(base, seq_len)
        pltpu.sync_copy(ids_hbm.at[wid], idx_v)

        def chunk_body(c, carry):
            pltpu.async_copy(table_hbm.at[idx_v.at[c]], rows_v, sem).wait()
            pltpu.sync_copy(pe_hbm.at[pl.ds(s_base + c * K, K)], pe_v)

            def row_body(i, carry2):
                def lane_body(j, carry3):
                    sl = pl.ds(j * LANES, LANES)
                    rows_v[i, sl] = rows_v[i, sl] + pe_v[i, sl]
                    return carry3
                return lax.fori_loop(0, D // LANES, lane_body, carry2)

            lax.fori_loop(0, K, row_body, 0)
            pltpu.sync_copy(rows_v, out_hbm.at[pl.ds(base + c * K, K)])
            return carry

        lax.fori_loop(0, n_chunks, chunk_body, 0)

    return k


def kernel(token_ids, table, pe):
    b, s = token_ids.shape
    n_rows = b * s
    rows_per_w = n_rows // NW
    n_chunks = rows_per_w // K
    ids = token_ids.astype(jnp.int32).reshape(NW, n_chunks, K)
    out = _emb_call(n_rows, s, n_chunks)(ids, table, pe[:s].astype(jnp.float32))
    return out.reshape(b, s, D)


# speedup vs baseline: 3.9193x; 3.9193x over previous
import functools

import jax
import jax.numpy as jnp
from jax import lax
from jax.experimental import pallas as pl
from jax.experimental.pallas import tpu as pltpu
from jax.experimental.pallas import tpu_sc as plsc

D = 1024
NC = 2
NS = 16
NW = NC * NS
K = 8
LANES = 16
NB = 4


@functools.lru_cache(maxsize=None)
def _emb_call(n_rows, seq_len, n_chunks):
    mesh = plsc.VectorSubcoreMesh(core_axis_name="c", subcore_axis_name="s")
    spw = seq_len // NW

    @functools.partial(
        pl.kernel,
        out_type=jax.ShapeDtypeStruct((n_rows, D), jnp.float32),
        mesh=mesh,
        scratch_types=[
            pltpu.VMEM((n_chunks, NB, K), jnp.int32),
            pltpu.VMEM((2, NB, K, D), jnp.float32),
            pltpu.VMEM((2, K, D), jnp.float32),
            pltpu.SemaphoreType.DMA,
            pltpu.SemaphoreType.DMA,
            pltpu.SemaphoreType.DMA,
            pltpu.SemaphoreType.DMA,
            pltpu.SemaphoreType.DMA,
            pltpu.SemaphoreType.DMA,
            pltpu.SemaphoreType.DMA,
            pltpu.SemaphoreType.DMA,
            pltpu.SemaphoreType.DMA,
        ],
    )
    def k(ids_hbm, table_hbm, pe_hbm, out_hbm, idx_v, rows_v, pe_v,
          sem_pe, sg0, sg1, sg2, sg3, ss0, ss1, ss2, ss3):
        wid = lax.axis_index("s") * NC + lax.axis_index("c")
        s0 = wid * spw
        sem_g = [sg0, sg1, sg2, sg3]
        sem_s = [ss0, ss1, ss2, ss3]

        pltpu.sync_copy(ids_hbm.at[wid], idx_v)

        def pe_copy(c, slot):
            return pltpu.make_async_copy(
                pe_hbm.at[pl.ds(s0 + c * K, K)], pe_v.at[slot], sem_pe)

        def gather(c, slot, b):
            return pltpu.make_async_copy(
                table_hbm.at[idx_v.at[c, b]], rows_v.at[slot, b], sem_g[b])

        def store(c, slot, b):
            return pltpu.make_async_copy(
                rows_v.at[slot, b],
                out_hbm.at[pl.ds(b * seq_len + s0 + c * K, K)], sem_s[b])

        pe_copy(0, 0).start()
        for b in range(NB):
            gather(0, 0, b).start()

        def chunk_body(c, carry):
            q = lax.rem(c, 2)
            qn = lax.rem(c + 1, 2)
            pe_copy(c, q).wait()

            @pl.when(c + 1 < n_chunks)
            def _():
                pe_copy(c + 1, qn).start()

            for b in range(NB):
                @pl.when(c > 0)
                def _():
                    store(c - 1, qn, b).wait()
                gather(c, q, b).wait()

                @pl.when(c + 1 < n_chunks)
                def _():
                    gather(c + 1, qn, b).start()

                jper = D // LANES

                @plsc.parallel_loop(0, K * jper, unroll=8)
                def _(i):
                    r = lax.shift_right_logical(i, 6)
                    j = lax.bitwise_and(i, jper - 1)
                    sl = pl.ds(j * LANES, LANES)
                    plsc.addupdate(rows_v.at[q, b, r, sl], pe_v[q, r, sl])
                store(c, q, b).start()
            return carry

        lax.fori_loop(0, n_chunks, chunk_body, 0)
        for b in range(NB):
            store(n_chunks - 1, lax.rem(n_chunks - 1, 2), b).wait()

    return k


def kernel(token_ids, table, pe):
    b, s = token_ids.shape
    n_rows = b * s
    spw = s // NW
    n_chunks = spw // K
    ids = jnp.transpose(
        token_ids.astype(jnp.int32).reshape(b, NW, n_chunks, K), (1, 2, 0, 3))
    out = _emb_call(n_rows, s, n_chunks)(ids, table, pe[:s].astype(jnp.float32))
    return out.reshape(b, s, D)
